# baseline (speedup 1.0000x reference)
import jax
import jax.numpy as jnp
from jax.experimental import pallas as pl
from jax.experimental.pallas import tpu as pltpu

_B, _S, _D = 16, 1024, 256
_ROWS = _B * _S
_HALF = _ROWS // 2
_SIZES = (256, 512, 1024, 2304, 2304, 1024, 512, 256)
_OFFS = tuple(sum(_SIZES[:k]) for k in range(len(_SIZES)))
_K = len(_SIZES)


def _copy_kernel(x_hbm, o_hbm, buf, insems, outsems):
    base = pl.program_id(0) * _HALF
    for k in range(_K):
        pltpu.make_async_copy(
            x_hbm.at[pl.ds(base + _OFFS[k], _SIZES[k]), :],
            buf.at[pl.ds(_OFFS[k], _SIZES[k]), :],
            insems.at[k],
        ).start()
    for k in range(_K):
        pltpu.make_async_copy(
            x_hbm.at[pl.ds(base + _OFFS[k], _SIZES[k]), :],
            buf.at[pl.ds(_OFFS[k], _SIZES[k]), :],
            insems.at[k],
        ).wait()
        pltpu.make_async_copy(
            buf.at[pl.ds(_OFFS[k], _SIZES[k]), :],
            o_hbm.at[pl.ds(base + _OFFS[k], _SIZES[k]), :],
            outsems.at[k],
        ).start()
    for k in range(_K):
        pltpu.make_async_copy(
            buf.at[pl.ds(_OFFS[k], _SIZES[k]), :],
            o_hbm.at[pl.ds(base + _OFFS[k], _SIZES[k]), :],
            outsems.at[k],
        ).wait()


def kernel(x, W):
    del W
    flat = x.reshape(_ROWS, _D)
    out = pl.pallas_call(
        _copy_kernel,
        grid=(2,),
        in_specs=[pl.BlockSpec(memory_space=pltpu.MemorySpace.HBM)],
        out_specs=pl.BlockSpec(memory_space=pltpu.MemorySpace.HBM),
        out_shape=jax.ShapeDtypeStruct((_ROWS, _D), x.dtype),
        scratch_shapes=[
            pltpu.VMEM((_HALF, _D), x.dtype),
            pltpu.SemaphoreType.DMA((_K,)),
            pltpu.SemaphoreType.DMA((_K,)),
        ],
        compiler_params=pltpu.CompilerParams(
            dimension_semantics=("parallel",),
        ),
    )(flat)
    return out.reshape(x.shape)


# speedup vs baseline: 1.1060x; 1.1060x over previous
import jax
import jax.numpy as jnp
from jax.experimental import pallas as pl
from jax.experimental.pallas import tpu as pltpu

_B, _S, _D = 16, 1024, 256
_ROWS = _B * _S
_SIZES = (512, 1024, 2048, 4608, 4608, 2048, 1024, 512)
_OFFS = tuple(sum(_SIZES[:k]) for k in range(len(_SIZES)))
_K = len(_SIZES)


def _copy_kernel(x_hbm, o_hbm, buf, insems, outsems):
    for k in range(_K):
        pltpu.make_async_copy(
            x_hbm.at[pl.ds(_OFFS[k], _SIZES[k]), :],
            buf.at[pl.ds(_OFFS[k], _SIZES[k]), :],
            insems.at[k],
        ).start()
    for k in range(_K):
        pltpu.make_async_copy(
            x_hbm.at[pl.ds(_OFFS[k], _SIZES[k]), :],
            buf.at[pl.ds(_OFFS[k], _SIZES[k]), :],
            insems.at[k],
        ).wait()
        pltpu.make_async_copy(
            buf.at[pl.ds(_OFFS[k], _SIZES[k]), :],
            o_hbm.at[pl.ds(_OFFS[k], _SIZES[k]), :],
            outsems.at[k],
        ).start()
    for k in range(_K):
        pltpu.make_async_copy(
            buf.at[pl.ds(_OFFS[k], _SIZES[k]), :],
            o_hbm.at[pl.ds(_OFFS[k], _SIZES[k]), :],
            outsems.at[k],
        ).wait()


def kernel(x, W):
    del W
    flat = x.reshape(_ROWS, _D)
    out = pl.pallas_call(
        _copy_kernel,
        in_specs=[pl.BlockSpec(memory_space=pltpu.MemorySpace.HBM)],
        out_specs=pl.BlockSpec(memory_space=pltpu.MemorySpace.HBM),
        out_shape=jax.ShapeDtypeStruct((_ROWS, _D), x.dtype),
        scratch_shapes=[
            pltpu.VMEM((_ROWS, _D), x.dtype),
            pltpu.SemaphoreType.DMA((_K,)),
            pltpu.SemaphoreType.DMA((_K,)),
        ],
    )(flat)
    return out.reshape(x.shape)
